# baseline (speedup 1.0000x reference)
import functools

import jax
import jax.numpy as jnp
from jax import lax
from jax.experimental import pallas as pl
from jax.experimental.pallas import tpu as pltpu
from jax.experimental.pallas import tpu_sc as plsc

L = 16


@functools.cache
def _make_kernel(nbins, NW, CH):
    NP = NW * CH
    BP = -(-(nbins + 1) // L) * L
    RS = 1
    while RS < BP:
        RS *= 2
    NV = CH // L
    BV = BP // L
    mesh = plsc.VectorSubcoreMesh(core_axis_name="c", subcore_axis_name="s")

    @functools.partial(
        pl.kernel,
        out_type=[
            jax.ShapeDtypeStruct((NP,), jnp.int32),
            jax.ShapeDtypeStruct((NP,), jnp.int32),
            jax.ShapeDtypeStruct((BP,), jnp.int32),
        ],
        mesh=mesh,
        scratch_types=[
            pltpu.VMEM((CH,), jnp.int32),
            pltpu.VMEM((CH,), jnp.int32),
            pltpu.VMEM((CH,), jnp.int32),
            pltpu.VMEM((CH,), jnp.int32),
            pltpu.VMEM((CH,), jnp.int32),
            pltpu.VMEM((RS,), jnp.int32),
            pltpu.VMEM((BP,), jnp.int32),
            pltpu.VMEM((BP,), jnp.int32),
            pltpu.VMEM((BP,), jnp.int32),
            pltpu.VMEM((BP,), jnp.int32),
            pltpu.VMEM((NW, RS), jnp.int32),
            pltpu.VMEM_SHARED((NW, RS), jnp.int32),
            pltpu.SemaphoreType.DMA,
            pltpu.SemaphoreType.DMA,
        ],
        compiler_params=pltpu.CompilerParams(needs_layout_passes=False),
    )
    def tkernel(c_hbm, r_hbm, colt_hbm, boff_hbm, offs_hbm,
                c_v, r_v, lr_v, rank_v, gi_v, cnt_v, pref_v, tot_v,
                start_v, offs_v, hist_l, hist_sh, sem0, sem1):
        cid = lax.axis_index("c")
        wid = lax.axis_index("s")

        @pl.when(cid == 0)
        def _():
            iota = lax.iota(jnp.int32, L)
            zeros = jnp.zeros((L,), jnp.int32)
            base = wid * CH
            pltpu.sync_copy(c_hbm.at[pl.ds(base, CH)], c_v)
            pltpu.sync_copy(r_hbm.at[pl.ds(base, CH)], r_v)
            for bv in range(RS // L):
                cnt_v[pl.ds(bv * L, L)] = zeros

            for v in range(NV):
                sl = pl.ds(v * L, L)
                c = c_v[sl]
                pe = zeros
                ea = zeros
                for k in range(1, L):
                    prev = plsc.load_gather(c_v, [v * L + ((iota - k) & (L - 1))])
                    pe = pe + jnp.where((iota >= k) & (prev == c), 1, 0)
                    nxt = plsc.load_gather(c_v, [v * L + ((iota + k) & (L - 1))])
                    ea = ea + jnp.where((iota < L - k) & (nxt == c), 1, 0)
                lr = plsc.load_gather(cnt_v, [c]) + pe
                lr_v[sl] = lr
                plsc.store_scatter(cnt_v, [c], lr + 1, mask=ea == 0)

            pltpu.sync_copy(cnt_v, hist_sh.at[wid])
            plsc.subcore_barrier()
            pltpu.sync_copy(hist_sh, hist_l)

            for bv in range(BV):
                sl = pl.ds(bv * L, L)

                def body(w, carry, sl=sl):
                    p, t = carry
                    h = hist_l[w, sl]
                    return p + h * jnp.where(w < wid, 1, 0), t + h

                p, t = lax.fori_loop(0, NW, body, (zeros, zeros))
                pref_v[sl] = p
                tot_v[sl] = t
            carry = jnp.int32(0)
            for bv in range(BV):
                sl = pl.ds(bv * L, L)
                t = tot_v[sl]
                excl = plsc.cumsum(t) - t + carry
                offs_v[sl] = excl
                start_v[sl] = excl + pref_v[sl]
                carry = carry + jnp.sum(t)

            @pl.when(wid == 0)
            def _():
                pltpu.sync_copy(offs_v, offs_hbm)

            for v in range(NV):
                sl = pl.ds(v * L, L)
                rank_v[sl] = plsc.load_gather(start_v, [c_v[sl]]) + lr_v[sl]
                gi_v[sl] = base + v * L + iota
            cp0 = pltpu.async_copy(r_v, colt_hbm.at[rank_v], sem0)
            cp1 = pltpu.async_copy(gi_v, boff_hbm.at[rank_v], sem1)
            cp0.wait()
            cp1.wait()

    return tkernel


def kernel(size, data, row_indices, column_indices, offsets):
    nnz = column_indices.shape[0]
    nbins = offsets.shape[0] - 1
    NW = 16
    CH = -(-nnz // (NW * L)) * L
    NP = NW * CH
    ci = column_indices.astype(jnp.int32)
    ri = row_indices.astype(jnp.int32)
    c_pad = jnp.concatenate([ci, jnp.full((NP - nnz,), nbins, jnp.int32)])
    r_pad = jnp.concatenate([ri, jnp.zeros((NP - nnz,), jnp.int32)])
    colt, boff, offs = _make_kernel(nbins, NW, CH)(c_pad, r_pad)
    offsets_t = offs[: nbins + 1]
    z = (size[1] // data.shape[1] - nbins).astype(jnp.int32) if hasattr(
        size[1], "dtype") else jnp.int32(size[1] // data.shape[1] - nbins)
    offsets_t = offsets_t.at[0].add(z)
    return colt[:nnz], offsets_t, boff[:nnz]
